# baseline (speedup 1.0000x reference)
import jax
import jax.numpy as jnp
from jax import lax
from jax.experimental import pallas as pl
from jax.experimental.pallas import tpu as pltpu
from jax.experimental.pallas import tpu_sc as plsc

N_ATOMS = 1000000
R = 8
G = 100000
L = 3
O = 4

NC = 2
NS = 16
NW = NC * NS
T = (R * G) // NW
K = 6240
NCH = 4
NGRP = K // 16
TAIL = T - NCH * K
TGRP = 3
TB = TGRP * 16
FLAG = 1 << 21
IDXMASK = FLAG - 1
B = 480


def _sc_body(tbl_hbm, pk0_hbm, pk1_hbm, pk2_hbm, part_hbm,
             pk0_v, pk1_v, pk2_v, idx_c, p_c, m_c,
             pkt0_v, pkt1_v, pkt2_v, idxt_v, pt_v, acc_v, sem):
    wid = lax.axis_index("s") * NC + lax.axis_index("c")
    g0 = wid * T

    iota = lax.iota(jnp.int32, 16)
    zero_v = jnp.zeros((16,), jnp.float32)
    zero_i = jnp.zeros((16,), jnp.int32)
    one_f = jnp.float32(1.0)

    def zinit(i, c):
        idx_c[pl.ds(i * 16, 16)] = zero_i
        return c
    lax.fori_loop(0, (L * K) // 16, zinit, 0)

    pkt0_v[pl.ds(TB - 16, 16)] = zero_i
    pkt1_v[pl.ds(TB - 16, 16)] = zero_i
    pkt2_v[pl.ds(TB - 16, 16)] = zero_i

    def chunk_body(ci, acc):
        b = g0 + ci * K
        pltpu.sync_copy(pk0_hbm.at[pl.ds(b, K)], pk0_v)
        pltpu.sync_copy(pk1_hbm.at[pl.ds(b, K)], pk1_v)
        pltpu.sync_copy(pk2_hbm.at[pl.ds(b, K)], pk2_v)

        def pass0(i, off):
            ds_ = pl.ds(i * 16, 16)
            v0 = pk0_v[ds_]
            v1 = pk1_v[ds_]
            v2 = pk2_v[ds_]
            keep = v0 < FLAG
            i0 = lax.shift_right_logical(v0 & IDXMASK, 1)
            i1 = lax.shift_right_logical(v1, 1)
            i2 = lax.shift_right_logical(v2, 1)
            mb = (v0 & 1) | ((v1 & 1) << 1) | ((v2 & 1) << 2)
            plsc.store_compressed(idx_c.at[pl.ds(off, 16)], i0, mask=keep)
            plsc.store_compressed(idx_c.at[pl.ds(K + off, 16)], i1, mask=keep)
            plsc.store_compressed(idx_c.at[pl.ds(2 * K + off, 16)], i2, mask=keep)
            plsc.store_compressed(m_c.at[pl.ds(off, 16)], mb, mask=keep)
            cnt = jnp.max(plsc.all_reduce_population_count(keep))
            return off + cnt

        n_c = lax.fori_loop(0, NGRP, pass0, jnp.int32(0))

        nblk = (n_c + (B - 1)) // B
        for l in range(L):
            def issue(k, c, l=l):
                pltpu.async_copy(
                    tbl_hbm.at[idx_c.at[pl.ds(l * K + k * B, B)]],
                    p_c.at[pl.ds(l * K + k * B, B)], sem)
                return c
            lax.fori_loop(0, nblk, issue, 0)

        def drain(k, c):
            pltpu.make_async_copy(tbl_hbm.at[pl.ds(0, B)],
                                  p_c.at[pl.ds(0, B)], sem).wait()
            return c
        lax.fori_loop(0, L * nblk, drain, 0)

        def pass2(j, a):
            ds_ = pl.ds(j * 16, 16)
            w = m_c[ds_]
            p0 = p_c[ds_]
            p1 = p_c[pl.ds(K + j * 16, 16)]
            p2 = p_c[pl.ds(2 * K + j * 16, 16)]
            sel0 = jnp.where((w & 1) == 1, p0, one_f - p0)
            sel1 = jnp.where((w & 2) == 2, p1, one_f - p1)
            sel2 = jnp.where((w & 4) == 4, p2, one_f - p2)
            prod = sel0 * sel1 * sel2
            valid = (j * 16 + iota) < n_c
            return a + jnp.where(valid, prod, zero_v)

        return lax.fori_loop(0, (n_c + 15) // 16, pass2, acc)

    acc = lax.fori_loop(0, NCH, chunk_body, zero_v)

    b = g0 + NCH * K
    pltpu.sync_copy(pk0_hbm.at[pl.ds(b, TAIL)], pkt0_v.at[pl.ds(0, TAIL)])
    pltpu.sync_copy(pk1_hbm.at[pl.ds(b, TAIL)], pkt1_v.at[pl.ds(0, TAIL)])
    pltpu.sync_copy(pk2_hbm.at[pl.ds(b, TAIL)], pkt2_v.at[pl.ds(0, TAIL)])
    pkt_refs = (pkt0_v, pkt1_v, pkt2_v)
    for j in range(TGRP):
        ds_ = pl.ds(j * 16, 16)
        for l in range(L):
            v = pkt_refs[l][ds_]
            idxt_v[pl.ds(l * TB + j * 16, 16)] = (
                lax.shift_right_logical(v & IDXMASK, 1))
    pltpu.async_copy(tbl_hbm.at[idxt_v], pt_v, sem).wait()
    for j in range(TGRP):
        ds_ = pl.ds(j * 16, 16)
        prod = jnp.ones((16,), jnp.float32)
        v0 = pkt0_v[ds_]
        for l in range(L):
            v = pkt_refs[l][ds_] if l else v0
            pv = pt_v[pl.ds(l * TB + j * 16, 16)]
            prod = prod * jnp.where((v & 1) == 1, pv, one_f - pv)
        valid = ((j * 16 + iota) < TAIL) & (v0 < FLAG)
        acc = acc + jnp.where(valid, prod, zero_v)

    acc_v[...] = acc
    pltpu.sync_copy(acc_v, part_hbm.at[wid])


_sc_kernel = pl.kernel(
    _sc_body,
    out_type=jax.ShapeDtypeStruct((NW, 16), jnp.float32),
    mesh=plsc.VectorSubcoreMesh(core_axis_name="c", subcore_axis_name="s"),
    compiler_params=pltpu.CompilerParams(needs_layout_passes=False),
    scratch_types=[
        pltpu.VMEM((K,), jnp.int32),
        pltpu.VMEM((K,), jnp.int32),
        pltpu.VMEM((K,), jnp.int32),
        pltpu.VMEM((L * K,), jnp.int32),
        pltpu.VMEM((L * K,), jnp.float32),
        pltpu.VMEM((K,), jnp.int32),
        pltpu.VMEM((TB,), jnp.int32),
        pltpu.VMEM((TB,), jnp.int32),
        pltpu.VMEM((TB,), jnp.int32),
        pltpu.VMEM((L * TB,), jnp.int32),
        pltpu.VMEM((L * TB,), jnp.float32),
        pltpu.VMEM((16,), jnp.float32),
        pltpu.SemaphoreType.DMA,
    ],
)


def _finish_body(part_ref, wrow_ref, cnt_ref, w_ref, out_ref):
    c0 = jnp.sum(w_ref[...] * (jnp.float32(G) + cnt_ref[...]))
    s = jnp.sum(part_ref[...] * wrow_ref[...])
    out_ref[...] = jnp.reshape(c0 - s, (1, 1))


_finish = pl.pallas_call(
    _finish_body,
    out_shape=jax.ShapeDtypeStruct((1, 1), jnp.float32),
)


def kernel(posterior_prob, latent_var_inds, latent_neg_mask, observed_neg_mask,
           observed_rule_cnts, rule_weights):
    base = latent_var_inds * 2 + latent_neg_mask
    obs_nz = jnp.sum(observed_neg_mask, axis=-1) != 0
    pk0 = (base[:, :, 0] + jnp.where(obs_nz, FLAG, 0)).reshape(-1)
    pk1 = base[:, :, 1].reshape(-1)
    pk2 = base[:, :, 2].reshape(-1)

    partials = _sc_kernel(posterior_prob, pk0, pk1, pk2)

    wrow = jnp.repeat(rule_weights[0], NW // R).reshape(NW, 1)
    out = _finish(partials, wrow, observed_rule_cnts.reshape(1, R),
                  rule_weights)
    return out.reshape(1)
